# baseline (speedup 1.0000x reference)
import jax
import jax.numpy as jnp
from jax import lax
from jax.experimental import pallas as pl
from jax.experimental.pallas import tpu as pltpu

_ROWS = 1024
_COLS = 100000
_K = 128
_NBUF = 8


def _gather_body(roi_ref, fmri_ref, out_ref, bufs, sems):
    def start(j):
        tc = pl.multiple_of((roi_ref[j] // 128) * 128, 128)
        slot = j % _NBUF
        pltpu.make_async_copy(
            fmri_ref.at[:, pl.ds(tc, 128)], bufs.at[slot], sems.at[slot]
        ).start()

    for j in range(_NBUF):
        start(j)

    out_ref[...] = jnp.zeros_like(out_ref)
    dst_lane = lax.broadcasted_iota(jnp.int32, (128, _K), 1)
    src_lane = lax.broadcasted_iota(jnp.int32, (128, _K), 0)

    def step(j, carry):
        slot = j % _NBUF
        pltpu.make_async_copy(
            fmri_ref.at[:, pl.ds(0, 128)], bufs.at[slot], sems.at[slot]
        ).wait()
        lane = roi_ref[j] % 128
        onehot = ((src_lane == lane) & (dst_lane == j)).astype(jnp.float32)
        out_ref[...] += jnp.dot(
            bufs[slot], onehot, preferred_element_type=jnp.float32)

        @pl.when(j + _NBUF < _K)
        def _():
            tc = pl.multiple_of((roi_ref[j + _NBUF] // 128) * 128, 128)
            pltpu.make_async_copy(
                fmri_ref.at[:, pl.ds(tc, 128)], bufs.at[slot], sems.at[slot]
            ).start()

        return carry

    lax.fori_loop(0, _K, step, 0)


def kernel(fmri, roi):
    return pl.pallas_call(
        _gather_body,
        out_shape=jax.ShapeDtypeStruct((_ROWS, _K), jnp.float32),
        in_specs=[
            pl.BlockSpec(memory_space=pltpu.SMEM),
            pl.BlockSpec(memory_space=pltpu.MemorySpace.HBM),
        ],
        out_specs=pl.BlockSpec(memory_space=pltpu.VMEM),
        scratch_shapes=[
            pltpu.VMEM((_NBUF, _ROWS, 128), jnp.float32),
            pltpu.SemaphoreType.DMA((_NBUF,)),
        ],
    )(roi, fmri)
